# baseline (speedup 1.0000x reference)
import functools

import jax
import jax.numpy as jnp
from jax import lax
from jax.experimental import pallas as pl
from jax.experimental.pallas import tpu as pltpu
from jax.experimental.pallas import tpu_sc as plsc

_N = 5000
_NPAD = 5120
_E = 80000
_P = 20000
_NC = 2
_NS = 16
_NW = _NC * _NS
_L = 16
_CHUNK = 128
_EFW = 128
_BM = 1000
_ZB = 4096


def _mm(x, w, b=None):
    m, k = x.shape
    f = w.shape[1]

    def kern_b(x_ref, w_ref, b_ref, o_ref):
        acc = lax.dot_general(x_ref[...], w_ref[...], (((1,), (0,)), ((), ())),
                              preferred_element_type=jnp.float32,
                              precision=lax.Precision.HIGHEST)
        o_ref[...] = acc + b_ref[...]

    def kern(x_ref, w_ref, o_ref):
        o_ref[...] = lax.dot_general(x_ref[...], w_ref[...],
                                     (((1,), (0,)), ((), ())),
                                     preferred_element_type=jnp.float32,
                                     precision=lax.Precision.HIGHEST)

    in_specs = [pl.BlockSpec((_BM, k), lambda i: (i, 0)),
                pl.BlockSpec((k, f), lambda i: (0, 0))]
    args = [x, w]
    if b is not None:
        in_specs.append(pl.BlockSpec((1, f), lambda i: (0, 0)))
        args.append(b.reshape(1, f))

    return pl.pallas_call(
        kern_b if b is not None else kern,
        grid=(m // _BM,),
        in_specs=in_specs,
        out_specs=pl.BlockSpec((_BM, f), lambda i: (i, 0)),
        out_shape=jax.ShapeDtypeStruct((m, f), jnp.float32),
    )(*args)


def _update(h, ws, bs, seg, sefc, wbe):
    m, dh = h.shape
    f = ws.shape[1]

    def kern(h_ref, ws_ref, bs_ref, s_ref, sefc_ref, wbe_ref, o_ref):
        sefc_v = sefc_ref[...]
        cnt = sefc_v[:, 16:17]
        num = s_ref[...] + lax.dot_general(
            sefc_v, wbe_ref[...], (((1,), (0,)), ((), ())),
            preferred_element_type=jnp.float32,
            precision=lax.Precision.HIGHEST)
        agg = num / jnp.maximum(cnt, 1.0)
        z = lax.dot_general(h_ref[...], ws_ref[...], (((1,), (0,)), ((), ())),
                            preferred_element_type=jnp.float32,
                            precision=lax.Precision.HIGHEST)
        z = jnp.maximum(z + bs_ref[...] + agg, 0.0)
        nrm = jnp.sqrt(jnp.sum(z * z, axis=1, keepdims=True))
        o_ref[...] = z / jnp.maximum(nrm, 1e-12)

    return pl.pallas_call(
        kern,
        grid=(m // _BM,),
        in_specs=[pl.BlockSpec((_BM, dh), lambda i: (i, 0)),
                  pl.BlockSpec((dh, f), lambda i: (0, 0)),
                  pl.BlockSpec((1, f), lambda i: (0, 0)),
                  pl.BlockSpec((_BM, f), lambda i: (i, 0)),
                  pl.BlockSpec((_BM, _EFW), lambda i: (i, 0)),
                  pl.BlockSpec((_EFW, f), lambda i: (0, 0))],
        out_specs=pl.BlockSpec((_BM, f), lambda i: (i, 0)),
        out_shape=jax.ShapeDtypeStruct((m, f), jnp.float32),
    )(h, ws, bs.reshape(1, f), seg, sefc, wbe)


def _sefred(sef_copies, cnt_copies):
    n = sef_copies.shape[1]
    bm = 1000

    def kern(s_ref, c_ref, o_ref):
        sef = jnp.sum(s_ref[...], axis=0)
        cnt = jnp.sum(c_ref[...], axis=0)
        o_ref[...] = jnp.concatenate(
            [sef, cnt, jnp.zeros((bm, _EFW - 17), jnp.float32)], axis=1)

    return pl.pallas_call(
        kern,
        grid=(n // bm,),
        in_specs=[pl.BlockSpec((16, bm, 16), lambda i: (0, i, 0)),
                  pl.BlockSpec((16, bm, 1), lambda i: (0, i, 0))],
        out_specs=pl.BlockSpec((bm, _EFW), lambda i: (i, 0)),
        out_shape=jax.ShapeDtypeStruct((n, _EFW), jnp.float32),
    )(sef_copies, cnt_copies.reshape(16, n, 1))


def _lanesum(part):
    npairs = part.shape[0]
    bm = 8000

    def kern(x_ref, o_ref):
        o_ref[...] = jnp.sum(x_ref[...], axis=1, keepdims=True)

    out = pl.pallas_call(
        kern,
        grid=(npairs // bm,),
        in_specs=[pl.BlockSpec((bm, _L), lambda i: (i, 0))],
        out_specs=pl.BlockSpec((bm, 1), lambda i: (i, 0)),
        out_shape=jax.ShapeDtypeStruct((npairs, 1), jnp.float32),
    )(part)
    return out.reshape(npairs)


def _zero_flat(accf, nwords):
    zeros16 = jnp.zeros((_L,), jnp.float32)

    def zf(j, _):
        accf[pl.ds(j * _L, _L)] = zeros16
        return 0
    lax.fori_loop(0, nwords // _L, zf, 0)


def _splat(v16, lane):
    return jnp.take_along_axis(v16, jnp.full((_L,), lane, jnp.int32), axis=0)


def _accumulate_chunk(rows_v, dst_v, accf, lanes):
    for g in range(_CHUNK // _L):
        d16 = dst_v[pl.ds(g * _L, _L)]
        for el in range(_L):
            idx = _splat(d16, el) * _L + lanes
            plsc.addupdate_scatter(accf, [idx], rows_v[g * _L + el, :])


@jax.jit
def _seg_full(table_t, src, dst):
    nch = _E // _CHUNK
    awords = _NPAD * _L

    mesh = plsc.VectorSubcoreMesh(core_axis_name="c", subcore_axis_name="s")

    @functools.partial(
        pl.kernel,
        out_type=jax.ShapeDtypeStruct((_NW * awords,), jnp.float32),
        mesh=mesh,
        compiler_params=pltpu.CompilerParams(use_tc_tiling_on_sc=False, needs_layout_passes=False),
        scratch_types=[
            pltpu.VMEM((_CHUNK,), jnp.int32),
            pltpu.VMEM((_CHUNK,), jnp.int32),
            pltpu.VMEM((_CHUNK, _L), jnp.float32),
            pltpu.VMEM((awords,), jnp.float32),
            pltpu.SemaphoreType.DMA,
        ],
    )
    def seg(t_hbm, s_hbm, d_hbm, out_hbm, src_v, dst_v, rows_v, accf,
            sem):
        cid = lax.axis_index("c")
        sid = lax.axis_index("s")
        wid = sid * _NC + cid
        _zero_flat(accf, awords)
        lanes = lax.iota(jnp.int32, _L)

        def chunk(k, _):
            off = k * _CHUNK
            pltpu.sync_copy(s_hbm.at[pl.ds(off, _CHUNK)], src_v)
            pltpu.sync_copy(d_hbm.at[pl.ds(off, _CHUNK)], dst_v)
            for j in range(_CHUNK // _L):
                sl = pl.ds(j * _L, _L)
                src_v[sl] = src_v[sl] + wid * _N
            pltpu.async_copy(t_hbm.at[src_v], rows_v, sem).wait()
            _accumulate_chunk(rows_v, dst_v, accf, lanes)
            return 0
        lax.fori_loop(0, nch, chunk, 0)
        pltpu.sync_copy(accf, out_hbm.at[pl.ds(wid * awords, awords)])

    return seg(table_t, src, dst)


@jax.jit
def _seg_pair2(ta_t, sa, da, tb_t, sb, db):
    nch = _E // _CHUNK
    awords = _NPAD * _L

    mesh = plsc.VectorSubcoreMesh(core_axis_name="c", subcore_axis_name="s")

    @functools.partial(
        pl.kernel,
        out_type=jax.ShapeDtypeStruct((_NW * awords,), jnp.float32),
        mesh=mesh,
        compiler_params=pltpu.CompilerParams(use_tc_tiling_on_sc=False, needs_layout_passes=False),
        scratch_types=[
            pltpu.VMEM((_CHUNK,), jnp.int32),
            pltpu.VMEM((_CHUNK,), jnp.int32),
            pltpu.VMEM((_CHUNK, _L), jnp.float32),
            pltpu.VMEM((awords,), jnp.float32),
            pltpu.SemaphoreType.DMA,
        ],
    )
    def seg(ta_hbm, sa_hbm, da_hbm, tb_hbm, sb_hbm, db_hbm, out_hbm,
            src_v, dst_v, rows_v, accf, sem):
        cid = lax.axis_index("c")
        sid = lax.axis_index("s")
        _zero_flat(accf, awords)
        lanes = lax.iota(jnp.int32, _L)

        def run(t_hbm, s_hbm, d_hbm):
            def chunk(k, _):
                off = k * _CHUNK
                pltpu.sync_copy(s_hbm.at[pl.ds(off, _CHUNK)], src_v)
                pltpu.sync_copy(d_hbm.at[pl.ds(off, _CHUNK)], dst_v)
                for j in range(_CHUNK // _L):
                    sl = pl.ds(j * _L, _L)
                    src_v[sl] = src_v[sl] + sid * _N
                pltpu.async_copy(t_hbm.at[src_v], rows_v, sem).wait()
                _accumulate_chunk(rows_v, dst_v, accf, lanes)
                return 0
            lax.fori_loop(0, nch, chunk, 0)

        @pl.when(cid == 0)
        def _():
            run(ta_hbm, sa_hbm, da_hbm)

        @pl.when(cid == 1)
        def _():
            run(tb_hbm, sb_hbm, db_hbm)

        wid = cid * _NS + sid
        pltpu.sync_copy(accf, out_hbm.at[pl.ds(wid * awords, awords)])

    return seg(ta_t, sa, da, tb_t, sb, db)


@jax.jit
def _efcnt_call(ef_cat, dst_cat):
    nch = _E // _CHUNK
    swords = _NPAD * _L
    blk = swords + _NPAD

    mesh = plsc.VectorSubcoreMesh(core_axis_name="c", subcore_axis_name="s")

    @functools.partial(
        pl.kernel,
        out_type=jax.ShapeDtypeStruct((_NW * blk,), jnp.float32),
        mesh=mesh,
        compiler_params=pltpu.CompilerParams(use_tc_tiling_on_sc=False, needs_layout_passes=False),
        scratch_types=[
            pltpu.VMEM((_CHUNK,), jnp.int32),
            pltpu.VMEM((_CHUNK, _L), jnp.float32),
            pltpu.VMEM((swords,), jnp.float32),
            pltpu.VMEM((_NPAD,), jnp.float32),
        ],
    )
    def efcnt(ef_hbm, d_hbm, out_hbm, dst_v, rows_v, accs, accc):
        cid = lax.axis_index("c")
        sid = lax.axis_index("s")
        wid = sid * _NC + cid
        grp = wid // _NS
        loc = wid % _NS
        _zero_flat(accs, swords)
        _zero_flat(accc, _NPAD)
        lanes = lax.iota(jnp.int32, _L)
        ones16 = jnp.ones((_L,), jnp.float32)
        mask0 = lanes == 0
        nch_s = (nch - loc + _NS - 1) // _NS

        def chunk(k, _):
            off = grp * _E + (loc + k * _NS) * _CHUNK
            pltpu.sync_copy(ef_hbm.at[pl.ds(off, _CHUNK)], rows_v)
            pltpu.sync_copy(d_hbm.at[pl.ds(off, _CHUNK)], dst_v)

            for g in range(_CHUNK // _L):
                d16 = dst_v[pl.ds(g * _L, _L)]
                for el in range(_L):
                    dsplat = _splat(d16, el)
                    plsc.addupdate_scatter(accs, [dsplat * _L + lanes],
                                           rows_v[g * _L + el, :])
                    plsc.addupdate_scatter(accc, [dsplat], ones16, mask=mask0)
            return 0
        lax.fori_loop(0, nch_s, chunk, 0)

        pltpu.sync_copy(accs, out_hbm.at[pl.ds(wid * blk, swords)])
        pltpu.sync_copy(accc, out_hbm.at[pl.ds(wid * blk + swords, _NPAD)])

    return efcnt(ef_cat, dst_cat)


@jax.jit
def _score_call(a, b, ia, ib):
    d = a.shape[1]
    npairs = ia.shape[0]
    ch = 80
    nch = npairs // ch

    mesh = plsc.VectorSubcoreMesh(core_axis_name="c", subcore_axis_name="s")

    @functools.partial(
        pl.kernel,
        out_type=jax.ShapeDtypeStruct((npairs * _L,), jnp.float32),
        mesh=mesh,
        scratch_types=[
            pltpu.VMEM((ch,), jnp.int32),
            pltpu.VMEM((ch,), jnp.int32),
            pltpu.VMEM((ch, d), jnp.float32),
            pltpu.VMEM((ch, d), jnp.float32),
            pltpu.VMEM((ch * _L,), jnp.float32),
            pltpu.SemaphoreType.DMA,
        ],
    )
    def score(a_hbm, b_hbm, ia_hbm, ib_hbm, out_hbm,
              ia_v, ib_v, ar_v, br_v, part_v, sem):
        cid = lax.axis_index("c")
        sid = lax.axis_index("s")
        wid = sid * _NC + cid
        nch_w = (nch - wid + _NW - 1) // _NW

        def chunk(k, _):
            off = (wid + k * _NW) * ch
            pltpu.sync_copy(ia_hbm.at[pl.ds(off, ch)], ia_v)
            pltpu.sync_copy(ib_hbm.at[pl.ds(off, ch)], ib_v)
            pltpu.async_copy(a_hbm.at[ia_v], ar_v, sem).wait()
            pltpu.async_copy(b_hbm.at[ib_v], br_v, sem).wait()

            def pair(p, _):
                acc = jnp.zeros((_L,), jnp.float32)
                for j in range(d // _L):
                    sl = pl.ds(j * _L, _L)
                    acc = acc + ar_v[p, sl] * br_v[p, sl]
                part_v[pl.ds(p * _L, _L)] = acc
                return 0
            lax.fori_loop(0, ch, pair, 0)
            pltpu.sync_copy(part_v, out_hbm.at[pl.ds(off * _L, ch * _L)])
            return 0
        lax.fori_loop(0, nch_w, chunk, 0)

    return score(a, b, ia, ib)


def _colmajor(p, nslices):
    n, w = p.shape
    assert w == nslices * _L
    return p.reshape(n, nslices, _L).transpose(1, 0, 2).reshape(nslices * n, _L)


def _seg_assemble(flat, nslices):
    r = flat.reshape(nslices, _NPAD, _L)[:, :_N]
    return r.transpose(1, 0, 2).reshape(_N, nslices * _L)


def _wbe(w_n, b_n):
    f = w_n.shape[1]
    dh = w_n.shape[0] - 16
    return jnp.concatenate(
        [w_n[dh:], b_n.reshape(1, f), jnp.zeros((_EFW - 17, f), jnp.float32)],
        axis=0)


def kernel(h_customer, h_product, ef_c2p, ef_p2c, W_ue, b_ue, W_ie, b_ie,
           W1n_c2p, b1n_c2p, W1s_p, b1s_p, W1n_p2c, b1n_p2c, W1s_c, b1s_c,
           W2n_c2p, b2n_c2p, W2s_p, b2s_p, W2n_p2c, b2n_p2c, W2s_c, b2s_c,
           c2p_edges, p2c_edges, pos_edges, neg_edges):
    src_cp, dst_cp = c2p_edges[0], c2p_edges[1]
    src_pc, dst_pc = p2c_edges[0], p2c_edges[1]

    hc = _mm(h_customer, W_ue, b_ue)
    hp = _mm(h_product, W_ie, b_ie)

    ef_cat = jnp.concatenate([ef_c2p, ef_p2c], axis=0)
    dst_cat = jnp.concatenate([dst_cp, dst_pc])
    efr = _efcnt_call(ef_cat, dst_cat).reshape(_NW, _NPAD * _L + _NPAD)
    sef = efr[:, :_NPAD * _L].reshape(_NW, _NPAD, _L)[:, :_N]
    cnt = efr[:, _NPAD * _L:][:, :_N]
    sefc_cp = _sefred(sef[:_NS], cnt[:_NS])
    sefc_pc = _sefred(sef[_NS:], cnt[_NS:])

    dh = hc.shape[1]

    p1c = _colmajor(_mm(hc, W1n_c2p[:dh]), 32)
    p1p = _colmajor(_mm(hp, W1n_p2c[:dh]), 32)
    s1p = _seg_assemble(_seg_full(p1c, src_cp, dst_cp), 32)
    s1c = _seg_assemble(_seg_full(p1p, src_pc, dst_pc), 32)
    hp1 = _update(hp, W1s_p, b1s_p, s1p, sefc_cp, _wbe(W1n_c2p, b1n_c2p))
    hc1 = _update(hc, W1s_c, b1s_c, s1c, sefc_pc, _wbe(W1n_p2c, b1n_p2c))

    p2ct = _colmajor(_mm(hc1, W2n_c2p[:dh]), 16)
    p2pt = _colmajor(_mm(hp1, W2n_p2c[:dh]), 16)
    s2 = _seg_pair2(p2ct, src_cp, dst_cp, p2pt, src_pc, dst_pc)
    s2 = s2.reshape(2, _NS * _NPAD * _L)
    s2p = _seg_assemble(s2[0], 16)
    s2c = _seg_assemble(s2[1], 16)
    hp2 = _update(hp1, W2s_p, b2s_p, s2p, sefc_cp, _wbe(W2n_c2p, b2n_c2p))
    hc2 = _update(hc1, W2s_c, b2s_c, s2c, sefc_pc, _wbe(W2n_p2c, b2n_p2c))

    ia = jnp.concatenate([pos_edges[0], neg_edges[0]])
    ib = jnp.concatenate([pos_edges[1], neg_edges[1]])
    part = _score_call(hc2, hp2, ia, ib)
    sc = _lanesum(part.reshape(2 * _P, _L))
    return (hc2, hp2, sc[:_P], sc[_P:])


# speedup vs baseline: 2.0178x; 2.0178x over previous
import functools

import jax
import jax.numpy as jnp
from jax import lax
from jax.experimental import pallas as pl
from jax.experimental.pallas import tpu as pltpu
from jax.experimental.pallas import tpu_sc as plsc

_N = 5000
_NPAD = 5120
_E = 80000
_P = 20000
_NC = 2
_NS = 16
_NW = _NC * _NS
_L = 16
_CHUNK = 128
_EFW = 128
_BM = 1000
_ZB = 4096


def _mm(x, w, b=None):
    m, k = x.shape
    f = w.shape[1]

    def kern_b(x_ref, w_ref, b_ref, o_ref):
        acc = lax.dot_general(x_ref[...], w_ref[...], (((1,), (0,)), ((), ())),
                              preferred_element_type=jnp.float32,
                              precision=lax.Precision.HIGHEST)
        o_ref[...] = acc + b_ref[...]

    def kern(x_ref, w_ref, o_ref):
        o_ref[...] = lax.dot_general(x_ref[...], w_ref[...],
                                     (((1,), (0,)), ((), ())),
                                     preferred_element_type=jnp.float32,
                                     precision=lax.Precision.HIGHEST)

    in_specs = [pl.BlockSpec((_BM, k), lambda i: (i, 0)),
                pl.BlockSpec((k, f), lambda i: (0, 0))]
    args = [x, w]
    if b is not None:
        in_specs.append(pl.BlockSpec((1, f), lambda i: (0, 0)))
        args.append(b.reshape(1, f))

    return pl.pallas_call(
        kern_b if b is not None else kern,
        grid=(m // _BM,),
        in_specs=in_specs,
        out_specs=pl.BlockSpec((_BM, f), lambda i: (i, 0)),
        out_shape=jax.ShapeDtypeStruct((m, f), jnp.float32),
    )(*args)


def _update(h, ws, bs, seg, sefc, wbe):
    m, dh = h.shape
    f = ws.shape[1]

    def kern(h_ref, ws_ref, bs_ref, s_ref, sefc_ref, wbe_ref, o_ref):
        sefc_v = sefc_ref[...]
        cnt = sefc_v[:, 16:17]
        num = s_ref[...] + lax.dot_general(
            sefc_v, wbe_ref[...], (((1,), (0,)), ((), ())),
            preferred_element_type=jnp.float32,
            precision=lax.Precision.HIGHEST)
        agg = num / jnp.maximum(cnt, 1.0)
        z = lax.dot_general(h_ref[...], ws_ref[...], (((1,), (0,)), ((), ())),
                            preferred_element_type=jnp.float32,
                            precision=lax.Precision.HIGHEST)
        z = jnp.maximum(z + bs_ref[...] + agg, 0.0)
        nrm = jnp.sqrt(jnp.sum(z * z, axis=1, keepdims=True))
        o_ref[...] = z / jnp.maximum(nrm, 1e-12)

    return pl.pallas_call(
        kern,
        grid=(m // _BM,),
        in_specs=[pl.BlockSpec((_BM, dh), lambda i: (i, 0)),
                  pl.BlockSpec((dh, f), lambda i: (0, 0)),
                  pl.BlockSpec((1, f), lambda i: (0, 0)),
                  pl.BlockSpec((_BM, f), lambda i: (i, 0)),
                  pl.BlockSpec((_BM, _EFW), lambda i: (i, 0)),
                  pl.BlockSpec((_EFW, f), lambda i: (0, 0))],
        out_specs=pl.BlockSpec((_BM, f), lambda i: (i, 0)),
        out_shape=jax.ShapeDtypeStruct((m, f), jnp.float32),
    )(h, ws, bs.reshape(1, f), seg, sefc, wbe)


def _sefred(sef_copies, cnt_copies):
    n = sef_copies.shape[1]
    bm = 1000

    def kern(s_ref, c_ref, o_ref):
        sef = jnp.sum(s_ref[...], axis=0)
        cnt = jnp.sum(c_ref[...], axis=0)
        o_ref[...] = jnp.concatenate(
            [sef, cnt, jnp.zeros((bm, _EFW - 17), jnp.float32)], axis=1)

    return pl.pallas_call(
        kern,
        grid=(n // bm,),
        in_specs=[pl.BlockSpec((16, bm, 16), lambda i: (0, i, 0)),
                  pl.BlockSpec((16, bm, 1), lambda i: (0, i, 0))],
        out_specs=pl.BlockSpec((bm, _EFW), lambda i: (i, 0)),
        out_shape=jax.ShapeDtypeStruct((n, _EFW), jnp.float32),
    )(sef_copies, cnt_copies.reshape(16, n, 1))


def _lanesum(part):
    npairs = part.shape[0]
    bm = 8000

    def kern(x_ref, o_ref):
        o_ref[...] = jnp.sum(x_ref[...], axis=1, keepdims=True)

    out = pl.pallas_call(
        kern,
        grid=(npairs // bm,),
        in_specs=[pl.BlockSpec((bm, _L), lambda i: (i, 0))],
        out_specs=pl.BlockSpec((bm, 1), lambda i: (i, 0)),
        out_shape=jax.ShapeDtypeStruct((npairs, 1), jnp.float32),
    )(part)
    return out.reshape(npairs)


def _zero_flat(accf, nwords):
    zeros16 = jnp.zeros((_L,), jnp.float32)

    def zf(j, _):
        accf[pl.ds(j * _L, _L)] = zeros16
        return 0
    lax.fori_loop(0, nwords // _L, zf, 0)


def _splat(v16, lane):
    return jnp.take_along_axis(v16, jnp.full((_L,), lane, jnp.int32), axis=0)


_SUP = 5
_SUPE = _SUP * _CHUNK
_NSUP = _E // _SUPE


def _accum_super(rows_b, dst_b, accf, lanes):
    def sub(s5, _):
        for g in range(_CHUNK // _L):
            d16 = dst_b[pl.ds(s5 * _CHUNK + g * _L, _L)]
            d16s = d16 * _L
            idxs = [_splat(d16s, el) + lanes for el in range(_L)]
            for el in range(_L):
                plsc.addupdate_scatter(
                    accf, [idxs[el]], rows_b[s5 * _CHUNK + g * _L + el, :])
        return 0
    lax.fori_loop(0, _SUP, sub, 0)


def _seg_engine(t_hbm, s_hbm, d_hbm, out_hbm, bufs, tbase, outslot):
    (src_a, dst_a, rows_a, src_b, dst_b, rows_b, accf, sem_a, sem_b) = bufs
    awords = _NPAD * _L
    lanes = lax.iota(jnp.int32, _L)

    def fire(k, src_x, dst_x, rows_x, sem_x):
        off = k * _SUPE
        pltpu.sync_copy(s_hbm.at[pl.ds(off, _SUPE)], src_x)
        pltpu.sync_copy(d_hbm.at[pl.ds(off, _SUPE)], dst_x)
        for j in range(_SUPE // _L):
            sl = pl.ds(j * _L, _L)
            src_x[sl] = src_x[sl] + tbase
        for j in range(_SUP):
            pltpu.async_copy(
                t_hbm.at[src_x.at[pl.ds(j * _CHUNK, _CHUNK)]],
                rows_x.at[pl.ds(j * _CHUNK, _CHUNK)], sem_x)

    def drain(src_x, rows_x, sem_x):
        for j in range(_SUP):
            pltpu.make_async_copy(
                t_hbm.at[src_x.at[pl.ds(j * _CHUNK, _CHUNK)]],
                rows_x.at[pl.ds(j * _CHUNK, _CHUNK)], sem_x).wait()

    _zero_flat(accf, awords)
    fire(0, src_a, dst_a, rows_a, sem_a)

    def body(i, _):
        fire(2 * i + 1, src_b, dst_b, rows_b, sem_b)
        drain(src_a, rows_a, sem_a)
        _accum_super(rows_a, dst_a, accf, lanes)
        fire(2 * i + 2, src_a, dst_a, rows_a, sem_a)
        drain(src_b, rows_b, sem_b)
        _accum_super(rows_b, dst_b, accf, lanes)
        return 0
    lax.fori_loop(0, (_NSUP - 1) // 2, body, 0)
    drain(src_a, rows_a, sem_a)
    _accum_super(rows_a, dst_a, accf, lanes)
    pltpu.sync_copy(accf, out_hbm.at[pl.ds(outslot * awords, awords)])


_SEG_SCRATCH = [
    pltpu.VMEM((_SUPE,), jnp.int32),
    pltpu.VMEM((_SUPE,), jnp.int32),
    pltpu.VMEM((_SUPE, _L), jnp.float32),
    pltpu.VMEM((_SUPE,), jnp.int32),
    pltpu.VMEM((_SUPE,), jnp.int32),
    pltpu.VMEM((_SUPE, _L), jnp.float32),
    pltpu.VMEM((_NPAD * _L,), jnp.float32),
    pltpu.SemaphoreType.DMA,
    pltpu.SemaphoreType.DMA,
]


@jax.jit
def _seg_full(table_t, src, dst):
    mesh = plsc.VectorSubcoreMesh(core_axis_name="c", subcore_axis_name="s")

    @functools.partial(
        pl.kernel,
        out_type=jax.ShapeDtypeStruct((_NW * _NPAD * _L,), jnp.float32),
        mesh=mesh,
        compiler_params=pltpu.CompilerParams(
            use_tc_tiling_on_sc=False, needs_layout_passes=False),
        scratch_types=list(_SEG_SCRATCH),
    )
    def seg(t_hbm, s_hbm, d_hbm, out_hbm, *bufs):
        cid = lax.axis_index("c")
        sid = lax.axis_index("s")
        wid = sid * _NC + cid
        _seg_engine(t_hbm, s_hbm, d_hbm, out_hbm, bufs, wid * _N, wid)

    return seg(table_t, src, dst)


@jax.jit
def _seg_pair2(ta_t, sa, da, tb_t, sb, db):
    mesh = plsc.VectorSubcoreMesh(core_axis_name="c", subcore_axis_name="s")

    @functools.partial(
        pl.kernel,
        out_type=jax.ShapeDtypeStruct((_NW * _NPAD * _L,), jnp.float32),
        mesh=mesh,
        compiler_params=pltpu.CompilerParams(
            use_tc_tiling_on_sc=False, needs_layout_passes=False),
        scratch_types=list(_SEG_SCRATCH),
    )
    def seg(ta_hbm, sa_hbm, da_hbm, tb_hbm, sb_hbm, db_hbm, out_hbm, *bufs):
        cid = lax.axis_index("c")
        sid = lax.axis_index("s")
        wid = cid * _NS + sid

        @pl.when(cid == 0)
        def _():
            _seg_engine(ta_hbm, sa_hbm, da_hbm, out_hbm, bufs, sid * _N, wid)

        @pl.when(cid == 1)
        def _():
            _seg_engine(tb_hbm, sb_hbm, db_hbm, out_hbm, bufs, sid * _N, wid)

    return seg(ta_t, sa, da, tb_t, sb, db)


@jax.jit
def _efcnt_call(ef_cat, dst_cat):
    nch = _E // _CHUNK
    swords = _NPAD * _L
    blk = swords + _NPAD

    mesh = plsc.VectorSubcoreMesh(core_axis_name="c", subcore_axis_name="s")

    @functools.partial(
        pl.kernel,
        out_type=jax.ShapeDtypeStruct((_NW * blk,), jnp.float32),
        mesh=mesh,
        compiler_params=pltpu.CompilerParams(use_tc_tiling_on_sc=False, needs_layout_passes=False),
        scratch_types=[
            pltpu.VMEM((_CHUNK,), jnp.int32),
            pltpu.VMEM((_CHUNK, _L), jnp.float32),
            pltpu.VMEM((swords,), jnp.float32),
            pltpu.VMEM((_NPAD,), jnp.float32),
        ],
    )
    def efcnt(ef_hbm, d_hbm, out_hbm, dst_v, rows_v, accs, accc):
        cid = lax.axis_index("c")
        sid = lax.axis_index("s")
        wid = sid * _NC + cid
        grp = wid // _NS
        loc = wid % _NS
        _zero_flat(accs, swords)
        _zero_flat(accc, _NPAD)
        lanes = lax.iota(jnp.int32, _L)
        ones16 = jnp.ones((_L,), jnp.float32)
        mask0 = lanes == 0
        nch_s = (nch - loc + _NS - 1) // _NS

        def chunk(k, _):
            off = grp * _E + (loc + k * _NS) * _CHUNK
            pltpu.sync_copy(ef_hbm.at[pl.ds(off, _CHUNK)], rows_v)
            pltpu.sync_copy(d_hbm.at[pl.ds(off, _CHUNK)], dst_v)

            for g in range(_CHUNK // _L):
                d16 = dst_v[pl.ds(g * _L, _L)]
                for el in range(_L):
                    dsplat = _splat(d16, el)
                    plsc.addupdate_scatter(accs, [dsplat * _L + lanes],
                                           rows_v[g * _L + el, :])
                    plsc.addupdate_scatter(accc, [dsplat], ones16, mask=mask0)
            return 0
        lax.fori_loop(0, nch_s, chunk, 0)

        pltpu.sync_copy(accs, out_hbm.at[pl.ds(wid * blk, swords)])
        pltpu.sync_copy(accc, out_hbm.at[pl.ds(wid * blk + swords, _NPAD)])

    return efcnt(ef_cat, dst_cat)


@jax.jit
def _score_call(a, b, ia, ib):
    d = a.shape[1]
    npairs = ia.shape[0]
    ch = 80
    nch = npairs // ch

    mesh = plsc.VectorSubcoreMesh(core_axis_name="c", subcore_axis_name="s")

    @functools.partial(
        pl.kernel,
        out_type=jax.ShapeDtypeStruct((npairs * _L,), jnp.float32),
        mesh=mesh,
        scratch_types=[
            pltpu.VMEM((ch,), jnp.int32),
            pltpu.VMEM((ch,), jnp.int32),
            pltpu.VMEM((ch, d), jnp.float32),
            pltpu.VMEM((ch, d), jnp.float32),
            pltpu.VMEM((ch * _L,), jnp.float32),
            pltpu.SemaphoreType.DMA,
        ],
    )
    def score(a_hbm, b_hbm, ia_hbm, ib_hbm, out_hbm,
              ia_v, ib_v, ar_v, br_v, part_v, sem):
        cid = lax.axis_index("c")
        sid = lax.axis_index("s")
        wid = sid * _NC + cid
        nch_w = (nch - wid + _NW - 1) // _NW

        def chunk(k, _):
            off = (wid + k * _NW) * ch
            pltpu.sync_copy(ia_hbm.at[pl.ds(off, ch)], ia_v)
            pltpu.sync_copy(ib_hbm.at[pl.ds(off, ch)], ib_v)
            pltpu.async_copy(a_hbm.at[ia_v], ar_v, sem).wait()
            pltpu.async_copy(b_hbm.at[ib_v], br_v, sem).wait()

            def pair(p, _):
                acc = jnp.zeros((_L,), jnp.float32)
                for j in range(d // _L):
                    sl = pl.ds(j * _L, _L)
                    acc = acc + ar_v[p, sl] * br_v[p, sl]
                part_v[pl.ds(p * _L, _L)] = acc
                return 0
            lax.fori_loop(0, ch, pair, 0)
            pltpu.sync_copy(part_v, out_hbm.at[pl.ds(off * _L, ch * _L)])
            return 0
        lax.fori_loop(0, nch_w, chunk, 0)

    return score(a, b, ia, ib)


def _colmajor(p, nslices):
    n, w = p.shape
    assert w == nslices * _L
    return p.reshape(n, nslices, _L).transpose(1, 0, 2).reshape(nslices * n, _L)


def _seg_assemble(flat, nslices):
    r = flat.reshape(nslices, _NPAD, _L)[:, :_N]
    return r.transpose(1, 0, 2).reshape(_N, nslices * _L)


def _wbe(w_n, b_n):
    f = w_n.shape[1]
    dh = w_n.shape[0] - 16
    return jnp.concatenate(
        [w_n[dh:], b_n.reshape(1, f), jnp.zeros((_EFW - 17, f), jnp.float32)],
        axis=0)


def kernel(h_customer, h_product, ef_c2p, ef_p2c, W_ue, b_ue, W_ie, b_ie,
           W1n_c2p, b1n_c2p, W1s_p, b1s_p, W1n_p2c, b1n_p2c, W1s_c, b1s_c,
           W2n_c2p, b2n_c2p, W2s_p, b2s_p, W2n_p2c, b2n_p2c, W2s_c, b2s_c,
           c2p_edges, p2c_edges, pos_edges, neg_edges):
    src_cp, dst_cp = c2p_edges[0], c2p_edges[1]
    src_pc, dst_pc = p2c_edges[0], p2c_edges[1]

    hc = _mm(h_customer, W_ue, b_ue)
    hp = _mm(h_product, W_ie, b_ie)

    ef_cat = jnp.concatenate([ef_c2p, ef_p2c], axis=0)
    dst_cat = jnp.concatenate([dst_cp, dst_pc])
    efr = _efcnt_call(ef_cat, dst_cat).reshape(_NW, _NPAD * _L + _NPAD)
    sef = efr[:, :_NPAD * _L].reshape(_NW, _NPAD, _L)[:, :_N]
    cnt = efr[:, _NPAD * _L:][:, :_N]
    sefc_cp = _sefred(sef[:_NS], cnt[:_NS])
    sefc_pc = _sefred(sef[_NS:], cnt[_NS:])

    dh = hc.shape[1]

    p1c = _colmajor(_mm(hc, W1n_c2p[:dh]), 32)
    p1p = _colmajor(_mm(hp, W1n_p2c[:dh]), 32)
    s1p = _seg_assemble(_seg_full(p1c, src_cp, dst_cp), 32)
    s1c = _seg_assemble(_seg_full(p1p, src_pc, dst_pc), 32)
    hp1 = _update(hp, W1s_p, b1s_p, s1p, sefc_cp, _wbe(W1n_c2p, b1n_c2p))
    hc1 = _update(hc, W1s_c, b1s_c, s1c, sefc_pc, _wbe(W1n_p2c, b1n_p2c))

    p2ct = _colmajor(_mm(hc1, W2n_c2p[:dh]), 16)
    p2pt = _colmajor(_mm(hp1, W2n_p2c[:dh]), 16)
    s2 = _seg_pair2(p2ct, src_cp, dst_cp, p2pt, src_pc, dst_pc)
    s2 = s2.reshape(2, _NS * _NPAD * _L)
    s2p = _seg_assemble(s2[0], 16)
    s2c = _seg_assemble(s2[1], 16)
    hp2 = _update(hp1, W2s_p, b2s_p, s2p, sefc_cp, _wbe(W2n_c2p, b2n_c2p))
    hc2 = _update(hc1, W2s_c, b2s_c, s2c, sefc_pc, _wbe(W2n_p2c, b2n_p2c))

    ia = jnp.concatenate([pos_edges[0], neg_edges[0]])
    ib = jnp.concatenate([pos_edges[1], neg_edges[1]])
    part = _score_call(hc2, hp2, ia, ib)
    sc = _lanesum(part.reshape(2 * _P, _L))
    return (hc2, hp2, sc[:_P], sc[_P:])


# speedup vs baseline: 2.0715x; 1.0266x over previous
import functools

import jax
import jax.numpy as jnp
from jax import lax
from jax.experimental import pallas as pl
from jax.experimental.pallas import tpu as pltpu
from jax.experimental.pallas import tpu_sc as plsc

_N = 5000
_NPAD = 5120
_E = 80000
_P = 20000
_NC = 2
_NS = 16
_NW = _NC * _NS
_L = 16
_CHUNK = 128
_EFW = 128
_BM = 1000
_ZB = 4096


def _mm(x, w, b=None):
    m, k = x.shape
    f = w.shape[1]

    def kern_b(x_ref, w_ref, b_ref, o_ref):
        acc = lax.dot_general(x_ref[...], w_ref[...], (((1,), (0,)), ((), ())),
                              preferred_element_type=jnp.float32,
                              precision=lax.Precision.DEFAULT)
        o_ref[...] = acc + b_ref[...]

    def kern(x_ref, w_ref, o_ref):
        o_ref[...] = lax.dot_general(x_ref[...], w_ref[...],
                                     (((1,), (0,)), ((), ())),
                                     preferred_element_type=jnp.float32,
                                     precision=lax.Precision.DEFAULT)

    in_specs = [pl.BlockSpec((_BM, k), lambda i: (i, 0)),
                pl.BlockSpec((k, f), lambda i: (0, 0))]
    args = [x, w]
    if b is not None:
        in_specs.append(pl.BlockSpec((1, f), lambda i: (0, 0)))
        args.append(b.reshape(1, f))

    return pl.pallas_call(
        kern_b if b is not None else kern,
        grid=(m // _BM,),
        in_specs=in_specs,
        out_specs=pl.BlockSpec((_BM, f), lambda i: (i, 0)),
        out_shape=jax.ShapeDtypeStruct((m, f), jnp.float32),
    )(*args)


def _update(h, ws, bs, seg, sefc, wbe):
    m, dh = h.shape
    f = ws.shape[1]

    def kern(h_ref, ws_ref, bs_ref, s_ref, sefc_ref, wbe_ref, o_ref):
        sefc_v = sefc_ref[...]
        cnt = sefc_v[:, 16:17]
        num = s_ref[...] + lax.dot_general(
            sefc_v, wbe_ref[...], (((1,), (0,)), ((), ())),
            preferred_element_type=jnp.float32,
            precision=lax.Precision.DEFAULT)
        agg = num / jnp.maximum(cnt, 1.0)
        z = lax.dot_general(h_ref[...], ws_ref[...], (((1,), (0,)), ((), ())),
                            preferred_element_type=jnp.float32,
                            precision=lax.Precision.DEFAULT)
        z = jnp.maximum(z + bs_ref[...] + agg, 0.0)
        nrm = jnp.sqrt(jnp.sum(z * z, axis=1, keepdims=True))
        o_ref[...] = z / jnp.maximum(nrm, 1e-12)

    return pl.pallas_call(
        kern,
        grid=(m // _BM,),
        in_specs=[pl.BlockSpec((_BM, dh), lambda i: (i, 0)),
                  pl.BlockSpec((dh, f), lambda i: (0, 0)),
                  pl.BlockSpec((1, f), lambda i: (0, 0)),
                  pl.BlockSpec((_BM, f), lambda i: (i, 0)),
                  pl.BlockSpec((_BM, _EFW), lambda i: (i, 0)),
                  pl.BlockSpec((_EFW, f), lambda i: (0, 0))],
        out_specs=pl.BlockSpec((_BM, f), lambda i: (i, 0)),
        out_shape=jax.ShapeDtypeStruct((m, f), jnp.float32),
    )(h, ws, bs.reshape(1, f), seg, sefc, wbe)


def _sefred(sef_copies, cnt_copies):
    n = sef_copies.shape[1]
    bm = 1000

    def kern(s_ref, c_ref, o_ref):
        sef = jnp.sum(s_ref[...], axis=0)
        cnt = jnp.sum(c_ref[...], axis=0)
        o_ref[...] = jnp.concatenate(
            [sef, cnt, jnp.zeros((bm, _EFW - 17), jnp.float32)], axis=1)

    return pl.pallas_call(
        kern,
        grid=(n // bm,),
        in_specs=[pl.BlockSpec((16, bm, 16), lambda i: (0, i, 0)),
                  pl.BlockSpec((16, bm, 1), lambda i: (0, i, 0))],
        out_specs=pl.BlockSpec((bm, _EFW), lambda i: (i, 0)),
        out_shape=jax.ShapeDtypeStruct((n, _EFW), jnp.float32),
    )(sef_copies, cnt_copies.reshape(16, n, 1))


def _lanesum(part):
    npairs = part.shape[0]
    bm = 8000

    def kern(x_ref, o_ref):
        o_ref[...] = jnp.sum(x_ref[...], axis=1, keepdims=True)

    out = pl.pallas_call(
        kern,
        grid=(npairs // bm,),
        in_specs=[pl.BlockSpec((bm, _L), lambda i: (i, 0))],
        out_specs=pl.BlockSpec((bm, 1), lambda i: (i, 0)),
        out_shape=jax.ShapeDtypeStruct((npairs, 1), jnp.float32),
    )(part)
    return out.reshape(npairs)


def _zero_flat(accf, nwords):
    zeros16 = jnp.zeros((_L,), jnp.float32)

    def zf(j, _):
        accf[pl.ds(j * _L, _L)] = zeros16
        return 0
    lax.fori_loop(0, nwords // _L, zf, 0)


def _splat(v16, lane):
    return jnp.take_along_axis(v16, jnp.full((_L,), lane, jnp.int32), axis=0)


_SUP = 5
_SUPE = _SUP * _CHUNK
_NSUP = _E // _SUPE


def _accum_super(rows_b, dst_b, accf, lanes):
    def sub(s5, _):
        for g in range(_CHUNK // _L):
            d16 = dst_b[pl.ds(s5 * _CHUNK + g * _L, _L)]
            d16s = d16 * _L
            idxs = [_splat(d16s, el) + lanes for el in range(_L)]
            for el in range(_L):
                plsc.addupdate_scatter(
                    accf, [idxs[el]], rows_b[s5 * _CHUNK + g * _L + el, :])
        return 0
    lax.fori_loop(0, _SUP, sub, 0)


def _seg_engine(t_hbm, s_hbm, d_hbm, out_hbm, bufs, tbase, outslot):
    (src_a, dst_a, rows_a, src_b, dst_b, rows_b, accf, sem_a, sem_b) = bufs
    awords = _NPAD * _L
    lanes = lax.iota(jnp.int32, _L)

    def fire(k, src_x, dst_x, rows_x, sem_x):
        off = k * _SUPE
        pltpu.sync_copy(s_hbm.at[pl.ds(off, _SUPE)], src_x)
        pltpu.sync_copy(d_hbm.at[pl.ds(off, _SUPE)], dst_x)
        for j in range(_SUPE // _L):
            sl = pl.ds(j * _L, _L)
            src_x[sl] = src_x[sl] + tbase
        for j in range(_SUP):
            pltpu.async_copy(
                t_hbm.at[src_x.at[pl.ds(j * _CHUNK, _CHUNK)]],
                rows_x.at[pl.ds(j * _CHUNK, _CHUNK)], sem_x)

    def drain(src_x, rows_x, sem_x):
        for j in range(_SUP):
            pltpu.make_async_copy(
                t_hbm.at[src_x.at[pl.ds(j * _CHUNK, _CHUNK)]],
                rows_x.at[pl.ds(j * _CHUNK, _CHUNK)], sem_x).wait()

    _zero_flat(accf, awords)
    fire(0, src_a, dst_a, rows_a, sem_a)

    def body(i, _):
        fire(2 * i + 1, src_b, dst_b, rows_b, sem_b)
        drain(src_a, rows_a, sem_a)
        _accum_super(rows_a, dst_a, accf, lanes)
        fire(2 * i + 2, src_a, dst_a, rows_a, sem_a)
        drain(src_b, rows_b, sem_b)
        _accum_super(rows_b, dst_b, accf, lanes)
        return 0
    lax.fori_loop(0, (_NSUP - 1) // 2, body, 0)
    drain(src_a, rows_a, sem_a)
    _accum_super(rows_a, dst_a, accf, lanes)
    pltpu.sync_copy(accf, out_hbm.at[pl.ds(outslot * awords, awords)])


_SEG_SCRATCH = [
    pltpu.VMEM((_SUPE,), jnp.int32),
    pltpu.VMEM((_SUPE,), jnp.int32),
    pltpu.VMEM((_SUPE, _L), jnp.float32),
    pltpu.VMEM((_SUPE,), jnp.int32),
    pltpu.VMEM((_SUPE,), jnp.int32),
    pltpu.VMEM((_SUPE, _L), jnp.float32),
    pltpu.VMEM((_NPAD * _L,), jnp.float32),
    pltpu.SemaphoreType.DMA,
    pltpu.SemaphoreType.DMA,
]


@jax.jit
def _seg_full(table_t, src, dst):
    mesh = plsc.VectorSubcoreMesh(core_axis_name="c", subcore_axis_name="s")

    @functools.partial(
        pl.kernel,
        out_type=jax.ShapeDtypeStruct((_NW * _NPAD * _L,), jnp.float32),
        mesh=mesh,
        compiler_params=pltpu.CompilerParams(
            use_tc_tiling_on_sc=False, needs_layout_passes=False),
        scratch_types=list(_SEG_SCRATCH),
    )
    def seg(t_hbm, s_hbm, d_hbm, out_hbm, *bufs):
        cid = lax.axis_index("c")
        sid = lax.axis_index("s")
        wid = sid * _NC + cid
        _seg_engine(t_hbm, s_hbm, d_hbm, out_hbm, bufs, wid * _N, wid)

    return seg(table_t, src, dst)


@jax.jit
def _seg_pair2(ta_t, sa, da, tb_t, sb, db):
    mesh = plsc.VectorSubcoreMesh(core_axis_name="c", subcore_axis_name="s")

    @functools.partial(
        pl.kernel,
        out_type=jax.ShapeDtypeStruct((_NW * _NPAD * _L,), jnp.float32),
        mesh=mesh,
        compiler_params=pltpu.CompilerParams(
            use_tc_tiling_on_sc=False, needs_layout_passes=False),
        scratch_types=list(_SEG_SCRATCH),
    )
    def seg(ta_hbm, sa_hbm, da_hbm, tb_hbm, sb_hbm, db_hbm, out_hbm, *bufs):
        cid = lax.axis_index("c")
        sid = lax.axis_index("s")
        wid = cid * _NS + sid

        @pl.when(cid == 0)
        def _():
            _seg_engine(ta_hbm, sa_hbm, da_hbm, out_hbm, bufs, sid * _N, wid)

        @pl.when(cid == 1)
        def _():
            _seg_engine(tb_hbm, sb_hbm, db_hbm, out_hbm, bufs, sid * _N, wid)

    return seg(ta_t, sa, da, tb_t, sb, db)


@jax.jit
def _efcnt_call(ef_cat, dst_cat):
    nch = _E // _CHUNK
    swords = _NPAD * _L
    blk = swords + _NPAD

    mesh = plsc.VectorSubcoreMesh(core_axis_name="c", subcore_axis_name="s")

    @functools.partial(
        pl.kernel,
        out_type=jax.ShapeDtypeStruct((_NW * blk,), jnp.float32),
        mesh=mesh,
        compiler_params=pltpu.CompilerParams(use_tc_tiling_on_sc=False, needs_layout_passes=False),
        scratch_types=[
            pltpu.VMEM((_CHUNK,), jnp.int32),
            pltpu.VMEM((_CHUNK, _L), jnp.float32),
            pltpu.VMEM((swords,), jnp.float32),
            pltpu.VMEM((_NPAD,), jnp.float32),
        ],
    )
    def efcnt(ef_hbm, d_hbm, out_hbm, dst_v, rows_v, accs, accc):
        cid = lax.axis_index("c")
        sid = lax.axis_index("s")
        wid = sid * _NC + cid
        grp = wid // _NS
        loc = wid % _NS
        _zero_flat(accs, swords)
        _zero_flat(accc, _NPAD)
        lanes = lax.iota(jnp.int32, _L)
        ones16 = jnp.ones((_L,), jnp.float32)
        mask0 = lanes == 0
        nch_s = (nch - loc + _NS - 1) // _NS

        def chunk(k, _):
            off = grp * _E + (loc + k * _NS) * _CHUNK
            pltpu.sync_copy(ef_hbm.at[pl.ds(off, _CHUNK)], rows_v)
            pltpu.sync_copy(d_hbm.at[pl.ds(off, _CHUNK)], dst_v)

            for g in range(_CHUNK // _L):
                d16 = dst_v[pl.ds(g * _L, _L)]
                for el in range(_L):
                    dsplat = _splat(d16, el)
                    plsc.addupdate_scatter(accs, [dsplat * _L + lanes],
                                           rows_v[g * _L + el, :])
                    plsc.addupdate_scatter(accc, [dsplat], ones16, mask=mask0)
            return 0
        lax.fori_loop(0, nch_s, chunk, 0)

        pltpu.sync_copy(accs, out_hbm.at[pl.ds(wid * blk, swords)])
        pltpu.sync_copy(accc, out_hbm.at[pl.ds(wid * blk + swords, _NPAD)])

    return efcnt(ef_cat, dst_cat)


@jax.jit
def _score_call(a, b, ia, ib):
    d = a.shape[1]
    npairs = ia.shape[0]
    ch = 80
    nch = npairs // ch

    mesh = plsc.VectorSubcoreMesh(core_axis_name="c", subcore_axis_name="s")

    @functools.partial(
        pl.kernel,
        out_type=jax.ShapeDtypeStruct((npairs * _L,), jnp.float32),
        mesh=mesh,
        scratch_types=[
            pltpu.VMEM((ch,), jnp.int32),
            pltpu.VMEM((ch,), jnp.int32),
            pltpu.VMEM((ch, d), jnp.float32),
            pltpu.VMEM((ch, d), jnp.float32),
            pltpu.VMEM((ch * _L,), jnp.float32),
            pltpu.SemaphoreType.DMA,
        ],
    )
    def score(a_hbm, b_hbm, ia_hbm, ib_hbm, out_hbm,
              ia_v, ib_v, ar_v, br_v, part_v, sem):
        cid = lax.axis_index("c")
        sid = lax.axis_index("s")
        wid = sid * _NC + cid
        nch_w = (nch - wid + _NW - 1) // _NW

        def chunk(k, _):
            off = (wid + k * _NW) * ch
            pltpu.sync_copy(ia_hbm.at[pl.ds(off, ch)], ia_v)
            pltpu.sync_copy(ib_hbm.at[pl.ds(off, ch)], ib_v)
            pltpu.async_copy(a_hbm.at[ia_v], ar_v, sem).wait()
            pltpu.async_copy(b_hbm.at[ib_v], br_v, sem).wait()

            def pair(p, _):
                acc = jnp.zeros((_L,), jnp.float32)
                for j in range(d // _L):
                    sl = pl.ds(j * _L, _L)
                    acc = acc + ar_v[p, sl] * br_v[p, sl]
                part_v[pl.ds(p * _L, _L)] = acc
                return 0
            lax.fori_loop(0, ch, pair, 0)
            pltpu.sync_copy(part_v, out_hbm.at[pl.ds(off * _L, ch * _L)])
            return 0
        lax.fori_loop(0, nch_w, chunk, 0)

    return score(a, b, ia, ib)


def _colmajor(p, nslices):
    n, w = p.shape
    assert w == nslices * _L
    return p.reshape(n, nslices, _L).transpose(1, 0, 2).reshape(nslices * n, _L)


def _seg_assemble(flat, nslices):
    r = flat.reshape(nslices, _NPAD, _L)[:, :_N]
    return r.transpose(1, 0, 2).reshape(_N, nslices * _L)


def _wbe(w_n, b_n):
    f = w_n.shape[1]
    dh = w_n.shape[0] - 16
    return jnp.concatenate(
        [w_n[dh:], b_n.reshape(1, f), jnp.zeros((_EFW - 17, f), jnp.float32)],
        axis=0)


def kernel(h_customer, h_product, ef_c2p, ef_p2c, W_ue, b_ue, W_ie, b_ie,
           W1n_c2p, b1n_c2p, W1s_p, b1s_p, W1n_p2c, b1n_p2c, W1s_c, b1s_c,
           W2n_c2p, b2n_c2p, W2s_p, b2s_p, W2n_p2c, b2n_p2c, W2s_c, b2s_c,
           c2p_edges, p2c_edges, pos_edges, neg_edges):
    src_cp, dst_cp = c2p_edges[0], c2p_edges[1]
    src_pc, dst_pc = p2c_edges[0], p2c_edges[1]

    hc = _mm(h_customer, W_ue, b_ue)
    hp = _mm(h_product, W_ie, b_ie)

    ef_cat = jnp.concatenate([ef_c2p, ef_p2c], axis=0)
    dst_cat = jnp.concatenate([dst_cp, dst_pc])
    efr = _efcnt_call(ef_cat, dst_cat).reshape(_NW, _NPAD * _L + _NPAD)
    sef = efr[:, :_NPAD * _L].reshape(_NW, _NPAD, _L)[:, :_N]
    cnt = efr[:, _NPAD * _L:][:, :_N]
    sefc_cp = _sefred(sef[:_NS], cnt[:_NS])
    sefc_pc = _sefred(sef[_NS:], cnt[_NS:])

    dh = hc.shape[1]

    p1c = _colmajor(_mm(hc, W1n_c2p[:dh]), 32)
    p1p = _colmajor(_mm(hp, W1n_p2c[:dh]), 32)
    s1p = _seg_assemble(_seg_full(p1c, src_cp, dst_cp), 32)
    s1c = _seg_assemble(_seg_full(p1p, src_pc, dst_pc), 32)
    hp1 = _update(hp, W1s_p, b1s_p, s1p, sefc_cp, _wbe(W1n_c2p, b1n_c2p))
    hc1 = _update(hc, W1s_c, b1s_c, s1c, sefc_pc, _wbe(W1n_p2c, b1n_p2c))

    p2ct = _colmajor(_mm(hc1, W2n_c2p[:dh]), 16)
    p2pt = _colmajor(_mm(hp1, W2n_p2c[:dh]), 16)
    s2 = _seg_pair2(p2ct, src_cp, dst_cp, p2pt, src_pc, dst_pc)
    s2 = s2.reshape(2, _NS * _NPAD * _L)
    s2p = _seg_assemble(s2[0], 16)
    s2c = _seg_assemble(s2[1], 16)
    hp2 = _update(hp1, W2s_p, b2s_p, s2p, sefc_cp, _wbe(W2n_c2p, b2n_c2p))
    hc2 = _update(hc1, W2s_c, b2s_c, s2c, sefc_pc, _wbe(W2n_p2c, b2n_p2c))

    ia = jnp.concatenate([pos_edges[0], neg_edges[0]])
    ib = jnp.concatenate([pos_edges[1], neg_edges[1]])
    part = _score_call(hc2, hp2, ia, ib)
    sc = _lanesum(part.reshape(2 * _P, _L))
    return (hc2, hp2, sc[:_P], sc[_P:])


# speedup vs baseline: 2.6613x; 1.2847x over previous
import functools

import jax
import jax.numpy as jnp
from jax import lax
from jax.experimental import pallas as pl
from jax.experimental.pallas import tpu as pltpu
from jax.experimental.pallas import tpu_sc as plsc

_N = 5000
_NPAD = 5120
_E = 80000
_P = 20000
_NC = 2
_NS = 16
_NW = _NC * _NS
_L = 16
_CHUNK = 128
_EFW = 128
_BM = 1000
_ZB = 4096


def _mm(x, w, b=None):
    m, k = x.shape
    f = w.shape[1]

    def kern_b(x_ref, w_ref, b_ref, o_ref):
        acc = lax.dot_general(x_ref[...], w_ref[...], (((1,), (0,)), ((), ())),
                              preferred_element_type=jnp.float32,
                              precision=lax.Precision.DEFAULT)
        o_ref[...] = acc + b_ref[...]

    def kern(x_ref, w_ref, o_ref):
        o_ref[...] = lax.dot_general(x_ref[...], w_ref[...],
                                     (((1,), (0,)), ((), ())),
                                     preferred_element_type=jnp.float32,
                                     precision=lax.Precision.DEFAULT)

    in_specs = [pl.BlockSpec((_BM, k), lambda i: (i, 0)),
                pl.BlockSpec((k, f), lambda i: (0, 0))]
    args = [x, w]
    if b is not None:
        in_specs.append(pl.BlockSpec((1, f), lambda i: (0, 0)))
        args.append(b.reshape(1, f))

    return pl.pallas_call(
        kern_b if b is not None else kern,
        grid=(m // _BM,),
        in_specs=in_specs,
        out_specs=pl.BlockSpec((_BM, f), lambda i: (i, 0)),
        out_shape=jax.ShapeDtypeStruct((m, f), jnp.float32),
    )(*args)


def _update(h, ws, bs, seg, sefc, wbe):
    m, dh = h.shape
    f = ws.shape[1]

    def kern(h_ref, ws_ref, bs_ref, s_ref, sefc_ref, wbe_ref, o_ref):
        sefc_v = sefc_ref[...]
        cnt = sefc_v[:, 16:17]
        num = s_ref[...] + lax.dot_general(
            sefc_v, wbe_ref[...], (((1,), (0,)), ((), ())),
            preferred_element_type=jnp.float32,
            precision=lax.Precision.DEFAULT)
        agg = num / jnp.maximum(cnt, 1.0)
        z = lax.dot_general(h_ref[...], ws_ref[...], (((1,), (0,)), ((), ())),
                            preferred_element_type=jnp.float32,
                            precision=lax.Precision.DEFAULT)
        z = jnp.maximum(z + bs_ref[...] + agg, 0.0)
        nrm = jnp.sqrt(jnp.sum(z * z, axis=1, keepdims=True))
        o_ref[...] = z / jnp.maximum(nrm, 1e-12)

    return pl.pallas_call(
        kern,
        grid=(m // _BM,),
        in_specs=[pl.BlockSpec((_BM, dh), lambda i: (i, 0)),
                  pl.BlockSpec((dh, f), lambda i: (0, 0)),
                  pl.BlockSpec((1, f), lambda i: (0, 0)),
                  pl.BlockSpec((_BM, f), lambda i: (i, 0)),
                  pl.BlockSpec((_BM, _EFW), lambda i: (i, 0)),
                  pl.BlockSpec((_EFW, f), lambda i: (0, 0))],
        out_specs=pl.BlockSpec((_BM, f), lambda i: (i, 0)),
        out_shape=jax.ShapeDtypeStruct((m, f), jnp.float32),
    )(h, ws, bs.reshape(1, f), seg, sefc, wbe)


def _sefred(sef_copies, cnt_copies):
    n = sef_copies.shape[1]
    bm = 1000

    def kern(s_ref, c_ref, o_ref):
        sef = jnp.sum(s_ref[...], axis=0)
        cnt = jnp.sum(c_ref[...], axis=0)
        o_ref[...] = jnp.concatenate(
            [sef, cnt, jnp.zeros((bm, _EFW - 17), jnp.float32)], axis=1)

    return pl.pallas_call(
        kern,
        grid=(n // bm,),
        in_specs=[pl.BlockSpec((16, bm, 16), lambda i: (0, i, 0)),
                  pl.BlockSpec((16, bm, 1), lambda i: (0, i, 0))],
        out_specs=pl.BlockSpec((bm, _EFW), lambda i: (i, 0)),
        out_shape=jax.ShapeDtypeStruct((n, _EFW), jnp.float32),
    )(sef_copies, cnt_copies.reshape(16, n, 1))


def _lanesum(part):
    npairs = part.shape[0]
    bm = 8000

    def kern(x_ref, o_ref):
        o_ref[...] = jnp.sum(x_ref[...], axis=1, keepdims=True)

    out = pl.pallas_call(
        kern,
        grid=(npairs // bm,),
        in_specs=[pl.BlockSpec((bm, _L), lambda i: (i, 0))],
        out_specs=pl.BlockSpec((bm, 1), lambda i: (i, 0)),
        out_shape=jax.ShapeDtypeStruct((npairs, 1), jnp.float32),
    )(part)
    return out.reshape(npairs)


def _zero_flat(accf, nwords):
    zeros16 = jnp.zeros((_L,), jnp.float32)

    def zf(j, _):
        accf[pl.ds(j * _L, _L)] = zeros16
        return 0
    lax.fori_loop(0, nwords // _L, zf, 0)


def _splat(v16, lane):
    return jnp.take_along_axis(v16, jnp.full((_L,), lane, jnp.int32), axis=0)


_SUP = 5
_SUPE = _SUP * _CHUNK
_NSUP = _E // _SUPE


def _accum_super(rows_b, dst_b, accf, lanes):
    def sub(s5, _):
        for g in range(_CHUNK // _L):
            base = s5 * _CHUNK + g * _L
            d16 = dst_b[pl.ds(base, _L)]
            d16s = d16 * _L
            idxs = [_splat(d16s, el) + lanes for el in range(_L)]
            vals = [rows_b[base + el, :] for el in range(_L)]
            for el in range(_L):
                plsc.addupdate_scatter(accf, [idxs[el]], vals[el])
        return 0
    lax.fori_loop(0, _SUP, sub, 0)


def _seg_engine(t_hbm, s_hbm, d_hbm, out_hbm, bufs, tbase, outslot):
    (src_a, dst_a, rows_a, src_b, dst_b, rows_b, accf, sem_a, sem_b) = bufs
    awords = _NPAD * _L
    lanes = lax.iota(jnp.int32, _L)

    def fire(k, src_x, dst_x, rows_x, sem_x):
        off = k * _SUPE
        pltpu.sync_copy(s_hbm.at[pl.ds(off, _SUPE)], src_x)
        pltpu.sync_copy(d_hbm.at[pl.ds(off, _SUPE)], dst_x)
        for j in range(_SUPE // _L):
            sl = pl.ds(j * _L, _L)
            src_x[sl] = src_x[sl] + tbase
        for j in range(_SUP):
            pltpu.async_copy(
                t_hbm.at[src_x.at[pl.ds(j * _CHUNK, _CHUNK)]],
                rows_x.at[pl.ds(j * _CHUNK, _CHUNK)], sem_x)

    def drain(src_x, rows_x, sem_x):
        for j in range(_SUP):
            pltpu.make_async_copy(
                t_hbm.at[src_x.at[pl.ds(j * _CHUNK, _CHUNK)]],
                rows_x.at[pl.ds(j * _CHUNK, _CHUNK)], sem_x).wait()

    _zero_flat(accf, awords)
    fire(0, src_a, dst_a, rows_a, sem_a)

    def body(i, _):
        fire(2 * i + 1, src_b, dst_b, rows_b, sem_b)
        drain(src_a, rows_a, sem_a)
        _accum_super(rows_a, dst_a, accf, lanes)
        fire(2 * i + 2, src_a, dst_a, rows_a, sem_a)
        drain(src_b, rows_b, sem_b)
        _accum_super(rows_b, dst_b, accf, lanes)
        return 0
    lax.fori_loop(0, (_NSUP - 1) // 2, body, 0)
    drain(src_a, rows_a, sem_a)
    _accum_super(rows_a, dst_a, accf, lanes)
    pltpu.sync_copy(accf, out_hbm.at[pl.ds(outslot * awords, awords)])


_SEG_SCRATCH = [
    pltpu.VMEM((_SUPE,), jnp.int32),
    pltpu.VMEM((_SUPE,), jnp.int32),
    pltpu.VMEM((_SUPE, _L), jnp.float32),
    pltpu.VMEM((_SUPE,), jnp.int32),
    pltpu.VMEM((_SUPE,), jnp.int32),
    pltpu.VMEM((_SUPE, _L), jnp.float32),
    pltpu.VMEM((_NPAD * _L,), jnp.float32),
    pltpu.SemaphoreType.DMA,
    pltpu.SemaphoreType.DMA,
]


@jax.jit
def _seg_full(table_t, src, dst):
    mesh = plsc.VectorSubcoreMesh(core_axis_name="c", subcore_axis_name="s")

    @functools.partial(
        pl.kernel,
        out_type=jax.ShapeDtypeStruct((_NW * _NPAD * _L,), jnp.float32),
        mesh=mesh,
        compiler_params=pltpu.CompilerParams(
            use_tc_tiling_on_sc=False, needs_layout_passes=False),
        scratch_types=list(_SEG_SCRATCH),
    )
    def seg(t_hbm, s_hbm, d_hbm, out_hbm, *bufs):
        cid = lax.axis_index("c")
        sid = lax.axis_index("s")
        wid = sid * _NC + cid
        _seg_engine(t_hbm, s_hbm, d_hbm, out_hbm, bufs, wid * _N, wid)

    return seg(table_t, src, dst)


@jax.jit
def _seg_pair2(ta_t, sa, da, tb_t, sb, db):
    mesh = plsc.VectorSubcoreMesh(core_axis_name="c", subcore_axis_name="s")

    @functools.partial(
        pl.kernel,
        out_type=jax.ShapeDtypeStruct((_NW * _NPAD * _L,), jnp.float32),
        mesh=mesh,
        compiler_params=pltpu.CompilerParams(
            use_tc_tiling_on_sc=False, needs_layout_passes=False),
        scratch_types=list(_SEG_SCRATCH),
    )
    def seg(ta_hbm, sa_hbm, da_hbm, tb_hbm, sb_hbm, db_hbm, out_hbm, *bufs):
        cid = lax.axis_index("c")
        sid = lax.axis_index("s")
        wid = cid * _NS + sid

        @pl.when(cid == 0)
        def _():
            _seg_engine(ta_hbm, sa_hbm, da_hbm, out_hbm, bufs, sid * _N, wid)

        @pl.when(cid == 1)
        def _():
            _seg_engine(tb_hbm, sb_hbm, db_hbm, out_hbm, bufs, sid * _N, wid)

    return seg(ta_t, sa, da, tb_t, sb, db)


@jax.jit
def _efcnt_call(ef_cat, dst_cat):
    nch = _E // _CHUNK
    swords = _NPAD * _L
    blk = swords + _NPAD

    mesh = plsc.VectorSubcoreMesh(core_axis_name="c", subcore_axis_name="s")

    @functools.partial(
        pl.kernel,
        out_type=jax.ShapeDtypeStruct((_NW * blk,), jnp.float32),
        mesh=mesh,
        compiler_params=pltpu.CompilerParams(use_tc_tiling_on_sc=False, needs_layout_passes=False),
        scratch_types=[
            pltpu.VMEM((_CHUNK,), jnp.int32),
            pltpu.VMEM((_CHUNK, _L), jnp.float32),
            pltpu.VMEM((swords,), jnp.float32),
            pltpu.VMEM((_NPAD,), jnp.float32),
        ],
    )
    def efcnt(ef_hbm, d_hbm, out_hbm, dst_v, rows_v, accs, accc):
        cid = lax.axis_index("c")
        sid = lax.axis_index("s")
        wid = sid * _NC + cid
        grp = wid // _NS
        loc = wid % _NS
        _zero_flat(accs, swords)
        _zero_flat(accc, _NPAD)
        lanes = lax.iota(jnp.int32, _L)
        ones16 = jnp.ones((_L,), jnp.float32)
        mask0 = lanes == 0
        nch_s = (nch - loc + _NS - 1) // _NS

        def chunk(k, _):
            off = grp * _E + (loc + k * _NS) * _CHUNK
            pltpu.sync_copy(ef_hbm.at[pl.ds(off, _CHUNK)], rows_v)
            pltpu.sync_copy(d_hbm.at[pl.ds(off, _CHUNK)], dst_v)

            for g in range(_CHUNK // _L):
                d16 = dst_v[pl.ds(g * _L, _L)]
                for el in range(_L):
                    dsplat = _splat(d16, el)
                    plsc.addupdate_scatter(accs, [dsplat * _L + lanes],
                                           rows_v[g * _L + el, :])
                    plsc.addupdate_scatter(accc, [dsplat], ones16, mask=mask0)
            return 0
        lax.fori_loop(0, nch_s, chunk, 0)

        pltpu.sync_copy(accs, out_hbm.at[pl.ds(wid * blk, swords)])
        pltpu.sync_copy(accc, out_hbm.at[pl.ds(wid * blk + swords, _NPAD)])

    return efcnt(ef_cat, dst_cat)


@jax.jit
def _score_call(a, b, ia, ib):
    d = a.shape[1]
    npairs = ia.shape[0]
    ch = 80
    nch = npairs // ch

    mesh = plsc.VectorSubcoreMesh(core_axis_name="c", subcore_axis_name="s")

    @functools.partial(
        pl.kernel,
        out_type=jax.ShapeDtypeStruct((npairs * _L,), jnp.float32),
        mesh=mesh,
        scratch_types=[
            pltpu.VMEM((ch,), jnp.int32),
            pltpu.VMEM((ch,), jnp.int32),
            pltpu.VMEM((ch, d), jnp.float32),
            pltpu.VMEM((ch, d), jnp.float32),
            pltpu.VMEM((ch * _L,), jnp.float32),
            pltpu.SemaphoreType.DMA,
        ],
    )
    def score(a_hbm, b_hbm, ia_hbm, ib_hbm, out_hbm,
              ia_v, ib_v, ar_v, br_v, part_v, sem):
        cid = lax.axis_index("c")
        sid = lax.axis_index("s")
        wid = sid * _NC + cid
        nch_w = (nch - wid + _NW - 1) // _NW

        def chunk(k, _):
            off = (wid + k * _NW) * ch
            pltpu.sync_copy(ia_hbm.at[pl.ds(off, ch)], ia_v)
            pltpu.sync_copy(ib_hbm.at[pl.ds(off, ch)], ib_v)
            pltpu.async_copy(a_hbm.at[ia_v], ar_v, sem).wait()
            pltpu.async_copy(b_hbm.at[ib_v], br_v, sem).wait()

            def pair(p, _):
                acc = jnp.zeros((_L,), jnp.float32)
                for j in range(d // _L):
                    sl = pl.ds(j * _L, _L)
                    acc = acc + ar_v[p, sl] * br_v[p, sl]
                part_v[pl.ds(p * _L, _L)] = acc
                return 0
            lax.fori_loop(0, ch, pair, 0)
            pltpu.sync_copy(part_v, out_hbm.at[pl.ds(off * _L, ch * _L)])
            return 0
        lax.fori_loop(0, nch_w, chunk, 0)

    return score(a, b, ia, ib)


def _colmajor(p, nslices):
    n, w = p.shape
    assert w == nslices * _L
    return p.reshape(n, nslices, _L).transpose(1, 0, 2).reshape(nslices * n, _L)


def _seg_assemble(flat, nslices):
    r = flat.reshape(nslices, _NPAD, _L)[:, :_N]
    return r.transpose(1, 0, 2).reshape(_N, nslices * _L)


def _wbe(w_n, b_n):
    f = w_n.shape[1]
    dh = w_n.shape[0] - 16
    return jnp.concatenate(
        [w_n[dh:], b_n.reshape(1, f), jnp.zeros((_EFW - 17, f), jnp.float32)],
        axis=0)


def kernel(h_customer, h_product, ef_c2p, ef_p2c, W_ue, b_ue, W_ie, b_ie,
           W1n_c2p, b1n_c2p, W1s_p, b1s_p, W1n_p2c, b1n_p2c, W1s_c, b1s_c,
           W2n_c2p, b2n_c2p, W2s_p, b2s_p, W2n_p2c, b2n_p2c, W2s_c, b2s_c,
           c2p_edges, p2c_edges, pos_edges, neg_edges):
    src_cp, dst_cp = c2p_edges[0], c2p_edges[1]
    src_pc, dst_pc = p2c_edges[0], p2c_edges[1]

    hc = _mm(h_customer, W_ue, b_ue)
    hp = _mm(h_product, W_ie, b_ie)

    ef_cat = jnp.concatenate([ef_c2p, ef_p2c], axis=0)
    dst_cat = jnp.concatenate([dst_cp, dst_pc])
    efr = _efcnt_call(ef_cat, dst_cat).reshape(_NW, _NPAD * _L + _NPAD)
    sef = efr[:, :_NPAD * _L].reshape(_NW, _NPAD, _L)[:, :_N]
    cnt = efr[:, _NPAD * _L:][:, :_N]
    sefc_cp = _sefred(sef[:_NS], cnt[:_NS])
    sefc_pc = _sefred(sef[_NS:], cnt[_NS:])

    dh = hc.shape[1]

    p1c = _colmajor(_mm(hc, W1n_c2p[:dh]), 32)
    p1p = _colmajor(_mm(hp, W1n_p2c[:dh]), 32)
    s1p = _seg_assemble(_seg_full(p1c, src_cp, dst_cp), 32)
    s1c = _seg_assemble(_seg_full(p1p, src_pc, dst_pc), 32)
    hp1 = _update(hp, W1s_p, b1s_p, s1p, sefc_cp, _wbe(W1n_c2p, b1n_c2p))
    hc1 = _update(hc, W1s_c, b1s_c, s1c, sefc_pc, _wbe(W1n_p2c, b1n_p2c))

    p2ct = _colmajor(_mm(hc1, W2n_c2p[:dh]), 16)
    p2pt = _colmajor(_mm(hp1, W2n_p2c[:dh]), 16)
    s2 = _seg_pair2(p2ct, src_cp, dst_cp, p2pt, src_pc, dst_pc)
    s2 = s2.reshape(2, _NS * _NPAD * _L)
    s2p = _seg_assemble(s2[0], 16)
    s2c = _seg_assemble(s2[1], 16)
    hp2 = _update(hp1, W2s_p, b2s_p, s2p, sefc_cp, _wbe(W2n_c2p, b2n_c2p))
    hc2 = _update(hc1, W2s_c, b2s_c, s2c, sefc_pc, _wbe(W2n_p2c, b2n_p2c))

    ia = jnp.concatenate([pos_edges[0], neg_edges[0]])
    ib = jnp.concatenate([pos_edges[1], neg_edges[1]])
    part = _score_call(hc2, hp2, ia, ib)
    sc = _lanesum(part.reshape(2 * _P, _L))
    return (hc2, hp2, sc[:_P], sc[_P:])


# speedup vs baseline: 2.6714x; 1.0038x over previous
import functools

import jax
import jax.numpy as jnp
from jax import lax
from jax.experimental import pallas as pl
from jax.experimental.pallas import tpu as pltpu
from jax.experimental.pallas import tpu_sc as plsc

_N = 5000
_NPAD = 5120
_E = 80000
_P = 20000
_NC = 2
_NS = 16
_NW = _NC * _NS
_L = 16
_CHUNK = 128
_EFW = 128
_BM = 1000
_ZB = 4096


def _mm(x, w, b=None):
    m, k = x.shape
    f = w.shape[1]

    def kern_b(x_ref, w_ref, b_ref, o_ref):
        acc = lax.dot_general(x_ref[...], w_ref[...], (((1,), (0,)), ((), ())),
                              preferred_element_type=jnp.float32,
                              precision=lax.Precision.DEFAULT)
        o_ref[...] = acc + b_ref[...]

    def kern(x_ref, w_ref, o_ref):
        o_ref[...] = lax.dot_general(x_ref[...], w_ref[...],
                                     (((1,), (0,)), ((), ())),
                                     preferred_element_type=jnp.float32,
                                     precision=lax.Precision.DEFAULT)

    in_specs = [pl.BlockSpec((_BM, k), lambda i: (i, 0)),
                pl.BlockSpec((k, f), lambda i: (0, 0))]
    args = [x, w]
    if b is not None:
        in_specs.append(pl.BlockSpec((1, f), lambda i: (0, 0)))
        args.append(b.reshape(1, f))

    return pl.pallas_call(
        kern_b if b is not None else kern,
        grid=(m // _BM,),
        in_specs=in_specs,
        out_specs=pl.BlockSpec((_BM, f), lambda i: (i, 0)),
        out_shape=jax.ShapeDtypeStruct((m, f), jnp.float32),
    )(*args)


def _update(h, ws, bs, seg, sefc, wbe):
    m, dh = h.shape
    f = ws.shape[1]

    def kern(h_ref, ws_ref, bs_ref, s_ref, sefc_ref, wbe_ref, o_ref):
        sefc_v = sefc_ref[...]
        cnt = sefc_v[:, 16:17]
        num = s_ref[...] + lax.dot_general(
            sefc_v, wbe_ref[...], (((1,), (0,)), ((), ())),
            preferred_element_type=jnp.float32,
            precision=lax.Precision.DEFAULT)
        agg = num / jnp.maximum(cnt, 1.0)
        z = lax.dot_general(h_ref[...], ws_ref[...], (((1,), (0,)), ((), ())),
                            preferred_element_type=jnp.float32,
                            precision=lax.Precision.DEFAULT)
        z = jnp.maximum(z + bs_ref[...] + agg, 0.0)
        nrm = jnp.sqrt(jnp.sum(z * z, axis=1, keepdims=True))
        o_ref[...] = z / jnp.maximum(nrm, 1e-12)

    return pl.pallas_call(
        kern,
        grid=(m // _BM,),
        in_specs=[pl.BlockSpec((_BM, dh), lambda i: (i, 0)),
                  pl.BlockSpec((dh, f), lambda i: (0, 0)),
                  pl.BlockSpec((1, f), lambda i: (0, 0)),
                  pl.BlockSpec((_BM, f), lambda i: (i, 0)),
                  pl.BlockSpec((_BM, _EFW), lambda i: (i, 0)),
                  pl.BlockSpec((_EFW, f), lambda i: (0, 0))],
        out_specs=pl.BlockSpec((_BM, f), lambda i: (i, 0)),
        out_shape=jax.ShapeDtypeStruct((m, f), jnp.float32),
    )(h, ws, bs.reshape(1, f), seg, sefc, wbe)


def _sefred(sef_copies, cnt_copies):
    n = sef_copies.shape[1]
    bm = 1000

    def kern(s_ref, c_ref, o_ref):
        sef = jnp.sum(s_ref[...], axis=0)
        cnt = jnp.sum(c_ref[...], axis=0)
        o_ref[...] = jnp.concatenate(
            [sef, cnt, jnp.zeros((bm, _EFW - 17), jnp.float32)], axis=1)

    return pl.pallas_call(
        kern,
        grid=(n // bm,),
        in_specs=[pl.BlockSpec((16, bm, 16), lambda i: (0, i, 0)),
                  pl.BlockSpec((16, bm, 1), lambda i: (0, i, 0))],
        out_specs=pl.BlockSpec((bm, _EFW), lambda i: (i, 0)),
        out_shape=jax.ShapeDtypeStruct((n, _EFW), jnp.float32),
    )(sef_copies, cnt_copies.reshape(16, n, 1))


def _lanesum(part):
    npairs = part.shape[0]
    bm = 8000

    def kern(x_ref, o_ref):
        o_ref[...] = jnp.sum(x_ref[...], axis=1, keepdims=True)

    out = pl.pallas_call(
        kern,
        grid=(npairs // bm,),
        in_specs=[pl.BlockSpec((bm, _L), lambda i: (i, 0))],
        out_specs=pl.BlockSpec((bm, 1), lambda i: (i, 0)),
        out_shape=jax.ShapeDtypeStruct((npairs, 1), jnp.float32),
    )(part)
    return out.reshape(npairs)


def _zero_flat(accf, nwords):
    zeros16 = jnp.zeros((_L,), jnp.float32)

    def zf(j, _):
        accf[pl.ds(j * _L, _L)] = zeros16
        return 0
    lax.fori_loop(0, nwords // _L, zf, 0)


def _splat(v16, lane):
    return jnp.take_along_axis(v16, jnp.full((_L,), lane, jnp.int32), axis=0)


_SUP = 5
_SUPE = _SUP * _CHUNK
_NSUP = _E // _SUPE


def _accum_super(rows_b, dst_b, accf, lanes):
    def sub(s5, _):
        for g in range(_CHUNK // _L):
            base = s5 * _CHUNK + g * _L
            d16 = dst_b[pl.ds(base, _L)]
            d16s = d16 * _L
            idxs = [_splat(d16s, el) + lanes for el in range(_L)]
            vals = [rows_b[base + el, :] for el in range(_L)]
            for el in range(_L):
                plsc.addupdate_scatter(accf, [idxs[el]], vals[el])
        return 0
    lax.fori_loop(0, _SUP, sub, 0)


def _seg_engine(t_hbm, s_hbm, d_hbm, out_hbm, bufs, tbase, outslot):
    (src_a, dst_a, rows_a, src_b, dst_b, rows_b, accf, sem_a, sem_b) = bufs
    awords = _NPAD * _L
    lanes = lax.iota(jnp.int32, _L)

    def fire(k, src_x, dst_x, rows_x, sem_x):
        off = k * _SUPE
        pltpu.sync_copy(s_hbm.at[pl.ds(off, _SUPE)], src_x)
        pltpu.sync_copy(d_hbm.at[pl.ds(off, _SUPE)], dst_x)
        for j in range(_SUPE // _L):
            sl = pl.ds(j * _L, _L)
            src_x[sl] = src_x[sl] + tbase
        for j in range(_SUP):
            pltpu.async_copy(
                t_hbm.at[src_x.at[pl.ds(j * _CHUNK, _CHUNK)]],
                rows_x.at[pl.ds(j * _CHUNK, _CHUNK)], sem_x)

    def drain(src_x, rows_x, sem_x):
        for j in range(_SUP):
            pltpu.make_async_copy(
                t_hbm.at[src_x.at[pl.ds(j * _CHUNK, _CHUNK)]],
                rows_x.at[pl.ds(j * _CHUNK, _CHUNK)], sem_x).wait()

    _zero_flat(accf, awords)
    fire(0, src_a, dst_a, rows_a, sem_a)

    def body(i, _):
        fire(2 * i + 1, src_b, dst_b, rows_b, sem_b)
        drain(src_a, rows_a, sem_a)
        _accum_super(rows_a, dst_a, accf, lanes)
        fire(2 * i + 2, src_a, dst_a, rows_a, sem_a)
        drain(src_b, rows_b, sem_b)
        _accum_super(rows_b, dst_b, accf, lanes)
        return 0
    lax.fori_loop(0, (_NSUP - 1) // 2, body, 0)
    drain(src_a, rows_a, sem_a)
    _accum_super(rows_a, dst_a, accf, lanes)
    pltpu.sync_copy(accf, out_hbm.at[pl.ds(outslot * awords, awords)])


_SEG_SCRATCH = [
    pltpu.VMEM((_SUPE,), jnp.int32),
    pltpu.VMEM((_SUPE,), jnp.int32),
    pltpu.VMEM((_SUPE, _L), jnp.float32),
    pltpu.VMEM((_SUPE,), jnp.int32),
    pltpu.VMEM((_SUPE,), jnp.int32),
    pltpu.VMEM((_SUPE, _L), jnp.float32),
    pltpu.VMEM((_NPAD * _L,), jnp.float32),
    pltpu.SemaphoreType.DMA,
    pltpu.SemaphoreType.DMA,
]


@jax.jit
def _seg_full(table_t, src, dst):
    mesh = plsc.VectorSubcoreMesh(core_axis_name="c", subcore_axis_name="s")

    @functools.partial(
        pl.kernel,
        out_type=jax.ShapeDtypeStruct((_NW * _NPAD * _L,), jnp.float32),
        mesh=mesh,
        compiler_params=pltpu.CompilerParams(
            use_tc_tiling_on_sc=False, needs_layout_passes=False),
        scratch_types=list(_SEG_SCRATCH),
    )
    def seg(t_hbm, s_hbm, d_hbm, out_hbm, *bufs):
        cid = lax.axis_index("c")
        sid = lax.axis_index("s")
        wid = sid * _NC + cid
        _seg_engine(t_hbm, s_hbm, d_hbm, out_hbm, bufs, wid * _N, wid)

    return seg(table_t, src, dst)


@jax.jit
def _seg_pair2(ta_t, sa, da, tb_t, sb, db):
    mesh = plsc.VectorSubcoreMesh(core_axis_name="c", subcore_axis_name="s")

    @functools.partial(
        pl.kernel,
        out_type=jax.ShapeDtypeStruct((_NW * _NPAD * _L,), jnp.float32),
        mesh=mesh,
        compiler_params=pltpu.CompilerParams(
            use_tc_tiling_on_sc=False, needs_layout_passes=False),
        scratch_types=list(_SEG_SCRATCH),
    )
    def seg(ta_hbm, sa_hbm, da_hbm, tb_hbm, sb_hbm, db_hbm, out_hbm, *bufs):
        cid = lax.axis_index("c")
        sid = lax.axis_index("s")
        wid = cid * _NS + sid

        @pl.when(cid == 0)
        def _():
            _seg_engine(ta_hbm, sa_hbm, da_hbm, out_hbm, bufs, sid * _N, wid)

        @pl.when(cid == 1)
        def _():
            _seg_engine(tb_hbm, sb_hbm, db_hbm, out_hbm, bufs, sid * _N, wid)

    return seg(ta_t, sa, da, tb_t, sb, db)


@jax.jit
def _efcnt_call(ef_cat, dst_cat):
    nch = _E // _CHUNK
    swords = _NPAD * _L
    blk = swords + _NPAD

    mesh = plsc.VectorSubcoreMesh(core_axis_name="c", subcore_axis_name="s")

    @functools.partial(
        pl.kernel,
        out_type=jax.ShapeDtypeStruct((_NW * blk,), jnp.float32),
        mesh=mesh,
        compiler_params=pltpu.CompilerParams(use_tc_tiling_on_sc=False, needs_layout_passes=False),
        scratch_types=[
            pltpu.VMEM((_CHUNK,), jnp.int32),
            pltpu.VMEM((_CHUNK, _L), jnp.float32),
            pltpu.VMEM((swords,), jnp.float32),
            pltpu.VMEM((_NPAD,), jnp.float32),
        ],
    )
    def efcnt(ef_hbm, d_hbm, out_hbm, dst_v, rows_v, accs, accc):
        cid = lax.axis_index("c")
        sid = lax.axis_index("s")
        wid = sid * _NC + cid
        grp = wid // _NS
        loc = wid % _NS
        _zero_flat(accs, swords)
        _zero_flat(accc, _NPAD)
        lanes = lax.iota(jnp.int32, _L)
        ones16 = jnp.ones((_L,), jnp.float32)
        mask0 = lanes == 0
        nch_s = (nch - loc + _NS - 1) // _NS

        def chunk(k, _):
            off = grp * _E + (loc + k * _NS) * _CHUNK
            pltpu.sync_copy(ef_hbm.at[pl.ds(off, _CHUNK)], rows_v)
            pltpu.sync_copy(d_hbm.at[pl.ds(off, _CHUNK)], dst_v)

            for g in range(_CHUNK // _L):
                d16 = dst_v[pl.ds(g * _L, _L)]
                splats = [_splat(d16, el) for el in range(_L)]
                idxs = [s * _L + lanes for s in splats]
                vals = [rows_v[g * _L + el, :] for el in range(_L)]
                for el in range(_L):
                    plsc.addupdate_scatter(accs, [idxs[el]], vals[el])
                for el in range(_L):
                    plsc.addupdate_scatter(accc, [splats[el]], ones16,
                                           mask=mask0)
            return 0
        lax.fori_loop(0, nch_s, chunk, 0)

        pltpu.sync_copy(accs, out_hbm.at[pl.ds(wid * blk, swords)])
        pltpu.sync_copy(accc, out_hbm.at[pl.ds(wid * blk + swords, _NPAD)])

    return efcnt(ef_cat, dst_cat)


@jax.jit
def _score_call(a, b, ia, ib):
    d = a.shape[1]
    npairs = ia.shape[0]
    ch = 80
    nch = npairs // ch

    mesh = plsc.VectorSubcoreMesh(core_axis_name="c", subcore_axis_name="s")

    @functools.partial(
        pl.kernel,
        out_type=jax.ShapeDtypeStruct((npairs * _L,), jnp.float32),
        mesh=mesh,
        scratch_types=[
            pltpu.VMEM((ch,), jnp.int32),
            pltpu.VMEM((ch,), jnp.int32),
            pltpu.VMEM((ch, d), jnp.float32),
            pltpu.VMEM((ch, d), jnp.float32),
            pltpu.VMEM((ch * _L,), jnp.float32),
            pltpu.SemaphoreType.DMA,
        ],
    )
    def score(a_hbm, b_hbm, ia_hbm, ib_hbm, out_hbm,
              ia_v, ib_v, ar_v, br_v, part_v, sem):
        cid = lax.axis_index("c")
        sid = lax.axis_index("s")
        wid = sid * _NC + cid
        nch_w = (nch - wid + _NW - 1) // _NW

        def chunk(k, _):
            off = (wid + k * _NW) * ch
            pltpu.sync_copy(ia_hbm.at[pl.ds(off, ch)], ia_v)
            pltpu.sync_copy(ib_hbm.at[pl.ds(off, ch)], ib_v)
            pltpu.async_copy(a_hbm.at[ia_v], ar_v, sem).wait()
            pltpu.async_copy(b_hbm.at[ib_v], br_v, sem).wait()

            def pair(p, _):
                acc = jnp.zeros((_L,), jnp.float32)
                for j in range(d // _L):
                    sl = pl.ds(j * _L, _L)
                    acc = acc + ar_v[p, sl] * br_v[p, sl]
                part_v[pl.ds(p * _L, _L)] = acc
                return 0
            lax.fori_loop(0, ch, pair, 0)
            pltpu.sync_copy(part_v, out_hbm.at[pl.ds(off * _L, ch * _L)])
            return 0
        lax.fori_loop(0, nch_w, chunk, 0)

    return score(a, b, ia, ib)


def _colmajor(p, nslices):
    n, w = p.shape
    assert w == nslices * _L
    return p.reshape(n, nslices, _L).transpose(1, 0, 2).reshape(nslices * n, _L)


def _seg_assemble(flat, nslices):
    r = flat.reshape(nslices, _NPAD, _L)[:, :_N]
    return r.transpose(1, 0, 2).reshape(_N, nslices * _L)


def _wbe(w_n, b_n):
    f = w_n.shape[1]
    dh = w_n.shape[0] - 16
    return jnp.concatenate(
        [w_n[dh:], b_n.reshape(1, f), jnp.zeros((_EFW - 17, f), jnp.float32)],
        axis=0)


def kernel(h_customer, h_product, ef_c2p, ef_p2c, W_ue, b_ue, W_ie, b_ie,
           W1n_c2p, b1n_c2p, W1s_p, b1s_p, W1n_p2c, b1n_p2c, W1s_c, b1s_c,
           W2n_c2p, b2n_c2p, W2s_p, b2s_p, W2n_p2c, b2n_p2c, W2s_c, b2s_c,
           c2p_edges, p2c_edges, pos_edges, neg_edges):
    src_cp, dst_cp = c2p_edges[0], c2p_edges[1]
    src_pc, dst_pc = p2c_edges[0], p2c_edges[1]

    hc = _mm(h_customer, W_ue, b_ue)
    hp = _mm(h_product, W_ie, b_ie)

    ef_cat = jnp.concatenate([ef_c2p, ef_p2c], axis=0)
    dst_cat = jnp.concatenate([dst_cp, dst_pc])
    efr = _efcnt_call(ef_cat, dst_cat).reshape(_NW, _NPAD * _L + _NPAD)
    sef = efr[:, :_NPAD * _L].reshape(_NW, _NPAD, _L)[:, :_N]
    cnt = efr[:, _NPAD * _L:][:, :_N]
    sefc_cp = _sefred(sef[:_NS], cnt[:_NS])
    sefc_pc = _sefred(sef[_NS:], cnt[_NS:])

    dh = hc.shape[1]

    p1c = _colmajor(_mm(hc, W1n_c2p[:dh]), 32)
    p1p = _colmajor(_mm(hp, W1n_p2c[:dh]), 32)
    s1p = _seg_assemble(_seg_full(p1c, src_cp, dst_cp), 32)
    s1c = _seg_assemble(_seg_full(p1p, src_pc, dst_pc), 32)
    hp1 = _update(hp, W1s_p, b1s_p, s1p, sefc_cp, _wbe(W1n_c2p, b1n_c2p))
    hc1 = _update(hc, W1s_c, b1s_c, s1c, sefc_pc, _wbe(W1n_p2c, b1n_p2c))

    p2ct = _colmajor(_mm(hc1, W2n_c2p[:dh]), 16)
    p2pt = _colmajor(_mm(hp1, W2n_p2c[:dh]), 16)
    s2 = _seg_pair2(p2ct, src_cp, dst_cp, p2pt, src_pc, dst_pc)
    s2 = s2.reshape(2, _NS * _NPAD * _L)
    s2p = _seg_assemble(s2[0], 16)
    s2c = _seg_assemble(s2[1], 16)
    hp2 = _update(hp1, W2s_p, b2s_p, s2p, sefc_cp, _wbe(W2n_c2p, b2n_c2p))
    hc2 = _update(hc1, W2s_c, b2s_c, s2c, sefc_pc, _wbe(W2n_p2c, b2n_p2c))

    ia = jnp.concatenate([pos_edges[0], neg_edges[0]])
    ib = jnp.concatenate([pos_edges[1], neg_edges[1]])
    part = _score_call(hc2, hp2, ia, ib)
    sc = _lanesum(part.reshape(2 * _P, _L))
    return (hc2, hp2, sc[:_P], sc[_P:])
